# baseline (speedup 1.0000x reference)
import functools
import jax
import jax.numpy as jnp
from jax import lax
from jax.experimental import pallas as pl
from jax.experimental.pallas import tpu as pltpu, tpu_sc as plsc

SEQ = 2048
D = 2048
CHUNK_ROWS = 8
NBUF = 4
LANES = 16
GROUP = 32


def _make_sc_kernel(nc, ns):
    nw = nc * ns
    rows_per_worker = SEQ // nw
    n_chunks = rows_per_worker // CHUNK_ROWS
    n_rounds = n_chunks // NBUF
    n_groups = D // (GROUP * LANES)

    mesh = plsc.VectorSubcoreMesh(core_axis_name="c", subcore_axis_name="s")

    @functools.partial(
        pl.kernel,
        out_type=(
            jax.ShapeDtypeStruct((SEQ, D), jnp.float32),
            jax.ShapeDtypeStruct((SEQ, D), jnp.float32),
        ),
        mesh=mesh,
        scratch_types=(
            pltpu.VMEM((D,), jnp.float32),
            pltpu.VMEM((D,), jnp.float32),
            [pltpu.VMEM((CHUNK_ROWS, D), jnp.float32) for _ in range(NBUF)],
            [pltpu.SemaphoreType.DMA for _ in range(NBUF)],
            [pltpu.SemaphoreType.DMA for _ in range(NBUF)],
        ),
    )
    def sc_kernel(ft_hbm, fi_hbm, ttab_hbm, itab_hbm, tout_hbm, iout_hbm,
                  ft_v, fi_v, bufs, in_sems, out_sems):
        wid = lax.axis_index("s") * nc + lax.axis_index("c")
        base_row = wid * rows_per_worker

        pltpu.sync_copy(ft_hbm, ft_v)
        pltpu.sync_copy(fi_hbm, fi_v)

        def compute(feat_v, b):
            buf = bufs[b]

            def gbody(g, _):
                base_col = g * GROUP * LANES
                fj = [feat_v[pl.ds(base_col + c * LANES, LANES)]
                      for c in range(GROUP)]

                @plsc.parallel_loop(0, CHUNK_ROWS, step=1)
                def rbody(r):
                    for c in range(GROUP):
                        sl = pl.ds(base_col + c * LANES, LANES)
                        buf[r, sl] = buf[r, sl] + fj[c]

                return 0

            lax.fori_loop(0, n_groups, gbody, 0)

        def run_modality(feat_v, tab, out):
            def start_in(chunk, b):
                pltpu.async_copy(
                    tab.at[pl.ds(base_row + chunk * CHUNK_ROWS, CHUNK_ROWS)],
                    bufs[b], in_sems[b])

            def start_out(chunk, b):
                pltpu.async_copy(
                    bufs[b],
                    out.at[pl.ds(base_row + chunk * CHUNK_ROWS, CHUNK_ROWS)],
                    out_sems[b])

            def wait_in(b):
                pltpu.make_async_copy(
                    tab.at[pl.ds(0, CHUNK_ROWS)], bufs[b], in_sems[b]).wait()

            def wait_out(b):
                pltpu.make_async_copy(
                    bufs[b], out.at[pl.ds(0, CHUNK_ROWS)], out_sems[b]).wait()

            for b in range(NBUF):
                start_in(b, b)

            def tbody(t, _):
                for b in range(NBUF):
                    wait_in(b)
                    compute(feat_v, b)
                    start_out(t * NBUF + b, b)

                    @pl.when(t + 1 < n_rounds)
                    def _():
                        wait_out(b)
                        start_in((t + 1) * NBUF + b, b)

                return 0

            lax.fori_loop(0, n_rounds, tbody, 0)
            for b in range(NBUF):
                wait_out(b)

        run_modality(ft_v, ttab_hbm, tout_hbm)
        run_modality(fi_v, itab_hbm, iout_hbm)

    return sc_kernel


def kernel(text, image, pos_table, text_pos_table, image_pos_table):
    del pos_table
    info = plsc.get_sparse_core_info()
    sc_k = _make_sc_kernel(info.num_cores, info.num_subcores)

    tout, iout = sc_k(text.reshape(-1), image.reshape(-1),
                      text_pos_table, image_pos_table)
    return (tout[None], iout[None])


# speedup vs baseline: 1.2728x; 1.2728x over previous
import functools
import jax
import jax.numpy as jnp
from jax import lax
from jax.experimental import pallas as pl
from jax.experimental.pallas import tpu as pltpu, tpu_sc as plsc

SEQ = 2048
D = 2048
CHUNK_ROWS = 16
NBUF = 3
LANES = 16
GROUP = 32
TC_BLOCK_ROWS = 256


def _make_sc_kernel(nc, ns):
    nw = nc * ns
    rows_per_worker = SEQ // nw
    n_chunks = rows_per_worker // CHUNK_ROWS
    n_groups = D // (GROUP * LANES)

    mesh = plsc.VectorSubcoreMesh(core_axis_name="c", subcore_axis_name="s")

    @functools.partial(
        pl.kernel,
        out_type=jax.ShapeDtypeStruct((SEQ, D), jnp.float32),
        mesh=mesh,
        scratch_types=(
            pltpu.VMEM((D,), jnp.float32),
            [pltpu.VMEM((CHUNK_ROWS, D), jnp.float32) for _ in range(NBUF)],
            [pltpu.SemaphoreType.DMA for _ in range(NBUF)],
            [pltpu.SemaphoreType.DMA for _ in range(NBUF)],
        ),
    )
    def sc_kernel(feat_hbm, tab_hbm, out_hbm, feat_v, bufs, in_sems, out_sems):
        wid = lax.axis_index("s") * nc + lax.axis_index("c")
        base_row = wid * rows_per_worker

        pltpu.sync_copy(feat_hbm, feat_v)

        rows = [base_row + ci * CHUNK_ROWS for ci in range(n_chunks)]
        nk = len(rows)

        def start_in(k):
            return pltpu.async_copy(
                tab_hbm.at[pl.ds(rows[k], CHUNK_ROWS)], bufs[k % NBUF],
                in_sems[k % NBUF])

        def start_out(k):
            return pltpu.async_copy(
                bufs[k % NBUF], out_hbm.at[pl.ds(rows[k], CHUNK_ROWS)],
                out_sems[k % NBUF])

        def compute(k):
            buf = bufs[k % NBUF]
            for g in range(n_groups):
                base_col = g * GROUP * LANES
                fj = [feat_v[pl.ds(base_col + c * LANES, LANES)]
                      for c in range(GROUP)]

                @plsc.parallel_loop(0, CHUNK_ROWS, step=1)
                def rbody(r):
                    for c in range(GROUP):
                        sl = pl.ds(base_col + c * LANES, LANES)
                        buf[r, sl] = buf[r, sl] + fj[c]

        in_fly = {0: start_in(0), 1: start_in(1)}
        out_fly = {}
        for k in range(nk):
            nxt = k + 2
            if nxt < nk:
                prev = nxt - NBUF
                if prev >= 0:
                    out_fly[prev].wait()
                in_fly[nxt] = start_in(nxt)
            in_fly[k].wait()
            compute(k)
            out_fly[k] = start_out(k)
        for k in range(max(0, nk - NBUF), nk):
            out_fly[k].wait()

    return sc_kernel


def _tc_add_kernel(feat_ref, tab_ref, out_ref):
    out_ref[...] = tab_ref[...] + feat_ref[...]


def _tc_add(feat2d, table):
    return pl.pallas_call(
        _tc_add_kernel,
        grid=(SEQ // TC_BLOCK_ROWS,),
        in_specs=[
            pl.BlockSpec((1, D), lambda i: (0, 0)),
            pl.BlockSpec((TC_BLOCK_ROWS, D), lambda i: (i, 0)),
        ],
        out_specs=pl.BlockSpec((TC_BLOCK_ROWS, D), lambda i: (i, 0)),
        out_shape=jax.ShapeDtypeStruct((SEQ, D), jnp.float32),
    )(feat2d, table[:SEQ])


def kernel(text, image, pos_table, text_pos_table, image_pos_table):
    del pos_table
    info = plsc.get_sparse_core_info()
    sc_k = _make_sc_kernel(info.num_cores, info.num_subcores)

    iout = sc_k(image.reshape(-1), image_pos_table)
    tout = _tc_add(text, text_pos_table)
    return (tout[None], iout[None])


# speedup vs baseline: 1.5612x; 1.2266x over previous
import functools
import jax
import jax.numpy as jnp
from jax import lax
from jax.experimental import pallas as pl
from jax.experimental.pallas import tpu as pltpu, tpu_sc as plsc

SEQ = 2048
D = 2048
CHUNK_ROWS = 16
NBUF = 3
LANES = 16
GROUP = 32
TC_BLOCK_ROWS = 256


def _make_sc_kernel(nc, ns):
    nw = nc * ns
    rows_per_worker = SEQ // nw
    n_chunks = rows_per_worker // CHUNK_ROWS
    n_groups = D // (GROUP * LANES)

    mesh = plsc.VectorSubcoreMesh(core_axis_name="c", subcore_axis_name="s")

    @functools.partial(
        pl.kernel,
        out_type=jax.ShapeDtypeStruct((SEQ, D), jnp.float32),
        mesh=mesh,
        scratch_types=(
            pltpu.VMEM((D,), jnp.float32),
            [pltpu.VMEM((CHUNK_ROWS, D), jnp.float32) for _ in range(NBUF)],
            [pltpu.SemaphoreType.DMA for _ in range(NBUF)],
            [pltpu.SemaphoreType.DMA for _ in range(NBUF)],
        ),
    )
    def sc_kernel(feat_hbm, tab_hbm, out_hbm, feat_v, bufs, in_sems, out_sems):
        wid = lax.axis_index("s") * nc + lax.axis_index("c")
        base_row = wid * rows_per_worker

        pltpu.sync_copy(feat_hbm, feat_v)

        rows = [base_row + ci * CHUNK_ROWS for ci in range(n_chunks)]
        nk = len(rows)

        def start_in(k):
            return pltpu.async_copy(
                tab_hbm.at[pl.ds(rows[k], CHUNK_ROWS)], bufs[k % NBUF],
                in_sems[k % NBUF])

        def start_out(k):
            return pltpu.async_copy(
                bufs[k % NBUF], out_hbm.at[pl.ds(rows[k], CHUNK_ROWS)],
                out_sems[k % NBUF])

        def compute(k):
            buf = bufs[k % NBUF]

            def gbody(g, _):
                base_col = g * GROUP * LANES
                fj = [feat_v[pl.ds(base_col + c * LANES, LANES)]
                      for c in range(GROUP)]

                @plsc.parallel_loop(0, CHUNK_ROWS, step=1)
                def rbody(r):
                    for c in range(GROUP):
                        sl = pl.ds(base_col + c * LANES, LANES)
                        buf[r, sl] = buf[r, sl] + fj[c]

                return 0

            lax.fori_loop(0, n_groups, gbody, 0)

        in_fly = {0: start_in(0), 1: start_in(1)}
        out_fly = {}
        for k in range(nk):
            nxt = k + 2
            if nxt < nk:
                prev = nxt - NBUF
                if prev >= 0:
                    out_fly[prev].wait()
                in_fly[nxt] = start_in(nxt)
            in_fly[k].wait()
            compute(k)
            out_fly[k] = start_out(k)
        for k in range(max(0, nk - NBUF), nk):
            out_fly[k].wait()

    return sc_kernel


def _tc_add_kernel(feat_ref, tab_ref, out_ref):
    out_ref[...] = tab_ref[...] + feat_ref[...]


def _tc_add(feat2d, table):
    return pl.pallas_call(
        _tc_add_kernel,
        grid=(SEQ // TC_BLOCK_ROWS,),
        in_specs=[
            pl.BlockSpec((1, D), lambda i: (0, 0)),
            pl.BlockSpec((TC_BLOCK_ROWS, D), lambda i: (i, 0)),
        ],
        out_specs=pl.BlockSpec((TC_BLOCK_ROWS, D), lambda i: (i, 0)),
        out_shape=jax.ShapeDtypeStruct((SEQ, D), jnp.float32),
    )(feat2d, table)


def kernel(text, image, pos_table, text_pos_table, image_pos_table):
    del pos_table
    info = plsc.get_sparse_core_info()
    sc_k = _make_sc_kernel(info.num_cores, info.num_subcores)

    iout = sc_k(image.reshape(-1), image_pos_table)
    tout = _tc_add(text, text_pos_table)
    return (tout[None], iout[None])


# speedup vs baseline: 1.5664x; 1.0033x over previous
import functools
import jax
import jax.numpy as jnp
from jax import lax
from jax.experimental import pallas as pl
from jax.experimental.pallas import tpu as pltpu, tpu_sc as plsc

SEQ = 2048
D = 2048
CHUNK_ROWS = 16
NBUF = 3
LANES = 16
GROUP = 32
TC_BLOCK_ROWS = 512


def _make_sc_kernel(nc, ns):
    nw = nc * ns
    rows_per_worker = SEQ // nw
    n_chunks = rows_per_worker // CHUNK_ROWS
    n_groups = D // (GROUP * LANES)

    mesh = plsc.VectorSubcoreMesh(core_axis_name="c", subcore_axis_name="s")

    @functools.partial(
        pl.kernel,
        out_type=jax.ShapeDtypeStruct((SEQ, D), jnp.float32),
        mesh=mesh,
        scratch_types=(
            pltpu.VMEM((D,), jnp.float32),
            [pltpu.VMEM((CHUNK_ROWS, D), jnp.float32) for _ in range(NBUF)],
            [pltpu.SemaphoreType.DMA for _ in range(NBUF)],
            [pltpu.SemaphoreType.DMA for _ in range(NBUF)],
        ),
    )
    def sc_kernel(feat_hbm, tab_hbm, out_hbm, feat_v, bufs, in_sems, out_sems):
        wid = lax.axis_index("s") * nc + lax.axis_index("c")
        base_row = wid * rows_per_worker

        pltpu.sync_copy(feat_hbm, feat_v)

        rows = [base_row + ci * CHUNK_ROWS for ci in range(n_chunks)]
        nk = len(rows)

        def start_in(k):
            return pltpu.async_copy(
                tab_hbm.at[pl.ds(rows[k], CHUNK_ROWS)], bufs[k % NBUF],
                in_sems[k % NBUF])

        def start_out(k):
            return pltpu.async_copy(
                bufs[k % NBUF], out_hbm.at[pl.ds(rows[k], CHUNK_ROWS)],
                out_sems[k % NBUF])

        def compute(k):
            buf = bufs[k % NBUF]

            def gbody(g, _):
                base_col = g * GROUP * LANES
                fj = [feat_v[pl.ds(base_col + c * LANES, LANES)]
                      for c in range(GROUP)]

                @plsc.parallel_loop(0, CHUNK_ROWS, step=1, unroll=4)
                def rbody(r):
                    for c in range(GROUP):
                        sl = pl.ds(base_col + c * LANES, LANES)
                        buf[r, sl] = buf[r, sl] + fj[c]

                return 0

            lax.fori_loop(0, n_groups, gbody, 0)

        in_fly = {0: start_in(0), 1: start_in(1)}
        out_fly = {}
        for k in range(nk):
            nxt = k + 2
            if nxt < nk:
                prev = nxt - NBUF
                if prev >= 0:
                    out_fly[prev].wait()
                in_fly[nxt] = start_in(nxt)
            in_fly[k].wait()
            compute(k)
            out_fly[k] = start_out(k)
        for k in range(max(0, nk - NBUF), nk):
            out_fly[k].wait()

    return sc_kernel


def _tc_add_kernel(feat_ref, tab_ref, out_ref):
    out_ref[...] = tab_ref[...] + feat_ref[...]


def _tc_add(feat2d, table):
    return pl.pallas_call(
        _tc_add_kernel,
        grid=(SEQ // TC_BLOCK_ROWS,),
        in_specs=[
            pl.BlockSpec((1, D), lambda i: (0, 0)),
            pl.BlockSpec((TC_BLOCK_ROWS, D), lambda i: (i, 0)),
        ],
        out_specs=pl.BlockSpec((TC_BLOCK_ROWS, D), lambda i: (i, 0)),
        out_shape=jax.ShapeDtypeStruct((SEQ, D), jnp.float32),
    )(feat2d, table)


def kernel(text, image, pos_table, text_pos_table, image_pos_table):
    del pos_table
    info = plsc.get_sparse_core_info()
    sc_k = _make_sc_kernel(info.num_cores, info.num_subcores)

    iout = sc_k(image.reshape(-1), image_pos_table)
    tout = _tc_add(text, text_pos_table)
    return (tout[None], iout[None])


# speedup vs baseline: 2.4195x; 1.5446x over previous
import jax
import jax.numpy as jnp
from jax.experimental import pallas as pl

SEQ = 2048
D = 2048
TC_BLOCK_ROWS = 256


def _tc_add_kernel(feat_ref, tab_ref, out_ref):
    out_ref[...] = tab_ref[...] + feat_ref[...]


def _tc_add(feat2d, table):
    return pl.pallas_call(
        _tc_add_kernel,
        grid=(SEQ // TC_BLOCK_ROWS,),
        in_specs=[
            pl.BlockSpec((1, D), lambda i: (0, 0)),
            pl.BlockSpec((TC_BLOCK_ROWS, D), lambda i: (i, 0)),
        ],
        out_specs=pl.BlockSpec((TC_BLOCK_ROWS, D), lambda i: (i, 0)),
        out_shape=jax.ShapeDtypeStruct((SEQ, D), jnp.float32),
    )(feat2d, table)


def kernel(text, image, pos_table, text_pos_table, image_pos_table):
    del pos_table
    tout = _tc_add(text, text_pos_table)
    iout = _tc_add(image, image_pos_table)
    return (tout[None], iout[None])
